# baseline (speedup 1.0000x reference)
import jax
import jax.numpy as jnp
from jax.experimental import pallas as pl
from jax.experimental.pallas import tpu as pltpu


def _pool_rows_kernel(x_ref, o_ref):
    o_ref[...] = jnp.max(x_ref[...], axis=-1, keepdims=True)


def kernel(x):
    N, C, H, W = x.shape
    rows = N * C
    hw = H * W

    x2d = x.reshape(rows, hw)

    row_tile = 1024
    grid = (pl.cdiv(rows, row_tile),)
    out2d = pl.pallas_call(
        _pool_rows_kernel,
        out_shape=jax.ShapeDtypeStruct((rows, 1), x.dtype),
        grid=grid,
        in_specs=[pl.BlockSpec((row_tile, hw), lambda i: (i, 0))],
        out_specs=pl.BlockSpec((row_tile, 1), lambda i: (i, 0)),
        compiler_params=pltpu.CompilerParams(
            dimension_semantics=("parallel",),
            vmem_limit_bytes=64 * 1024 * 1024,
        ),
    )(x2d)
    return out2d.reshape(N, C, 1, 1)
